# baseline (device time: 47193 ns/iter reference)
import jax
import jax.numpy as jnp
from jax import lax
from jax.experimental import pallas as pl
from jax.experimental.pallas import tpu as pltpu

N_DEV = 4
M_BLK = 2048 // N_DEV


def kernel(x, w_mat):
    m, k_per = x.shape
    _, n = w_mat.shape
    nh = n // 2

    def body(x_ref, w_ref, out_ref, xbf_ref,
             s1A, r1A, s1B, r1B, s2A, r2A, s2B, r2B,
             send_sems, recv_sems):
        p = lax.axis_index("i")
        p1 = jnp.bitwise_xor(p, 1)
        p2 = 3 - p

        barrier_sem = pltpu.get_barrier_semaphore()
        for nbr in [p1, p2]:
            pl.semaphore_signal(
                barrier_sem, inc=1,
                device_id=(nbr,), device_id_type=pl.DeviceIdType.MESH,
            )
        pl.semaphore_wait(barrier_sem, 2)

        xbf_ref[:, :] = x_ref[:, :].astype(jnp.bfloat16)
        w_bf = w_ref[:, :].astype(jnp.bfloat16)

        def ph(c, half):
            xc = xbf_ref[pl.ds(c * M_BLK, M_BLK), :]
            wc = w_bf[:, half * nh:(half + 1) * nh]
            return jnp.dot(xc, wc, preferred_element_type=jnp.float32)

        def mk(src, dst, si, ri, dev):
            return pltpu.make_async_remote_copy(
                src_ref=src, dst_ref=dst,
                send_sem=send_sems.at[si], recv_sem=recv_sems.at[ri],
                device_id=(dev,), device_id_type=pl.DeviceIdType.MESH,
            )

        bA0 = 3 - p1
        bB0 = jnp.bitwise_xor(p2, 1)

        s1A[0] = ph(bA0, 0).astype(jnp.bfloat16)
        d1A0 = mk(s1A.at[0], r1A.at[0], 0, 0, p1)
        d1A0.start()
        s1B[0] = ph(bB0, 1).astype(jnp.bfloat16)
        d1B0 = mk(s1B.at[0], r1B.at[0], 2, 2, p2)
        d1B0.start()
        s1A[1] = ph(p1, 0).astype(jnp.bfloat16)
        d1A1 = mk(s1A.at[1], r1A.at[1], 1, 1, p1)
        d1A1.start()
        s1B[1] = ph(p2, 1).astype(jnp.bfloat16)
        d1B1 = mk(s1B.at[1], r1B.at[1], 3, 3, p2)
        d1B1.start()

        pA_fwd = ph(p2, 0)
        pB_fwd = ph(p1, 1)

        d1A0.wait_recv()
        s2A[:, :] = (r1A[0].astype(jnp.float32) + pA_fwd).astype(jnp.bfloat16)
        d2A = mk(s2A, r2A, 4, 4, p2)
        d2A.start()
        d1B0.wait_recv()
        s2B[:, :] = (r1B[0].astype(jnp.float32) + pB_fwd).astype(jnp.bfloat16)
        d2B = mk(s2B, r2B, 5, 5, p1)
        d2B.start()

        pA_own = ph(p, 0)
        pB_own = ph(p, 1)

        d1A1.wait_recv()
        accA = pA_own + r1A[1].astype(jnp.float32)
        d1B1.wait_recv()
        accB = pB_own + r1B[1].astype(jnp.float32)
        d2A.wait_recv()
        out_ref[:, 0:nh] = accA + r2A[:, :].astype(jnp.float32)
        d2B.wait_recv()
        out_ref[:, nh:n] = accB + r2B[:, :].astype(jnp.float32)

        for d in [d1A0, d1A1, d1B0, d1B1, d2A, d2B]:
            d.wait_send()

    blk_half = (M_BLK, nh)
    return pl.pallas_call(
        body,
        out_shape=jax.ShapeDtypeStruct((M_BLK, n), jnp.float32),
        in_specs=[
            pl.BlockSpec(memory_space=pltpu.VMEM),
            pl.BlockSpec(memory_space=pltpu.VMEM),
        ],
        out_specs=pl.BlockSpec(memory_space=pltpu.VMEM),
        scratch_shapes=[
            pltpu.VMEM((m, k_per), jnp.bfloat16),
            pltpu.VMEM((2,) + blk_half, jnp.bfloat16),
            pltpu.VMEM((2,) + blk_half, jnp.bfloat16),
            pltpu.VMEM((2,) + blk_half, jnp.bfloat16),
            pltpu.VMEM((2,) + blk_half, jnp.bfloat16),
            pltpu.VMEM(blk_half, jnp.bfloat16),
            pltpu.VMEM(blk_half, jnp.bfloat16),
            pltpu.VMEM(blk_half, jnp.bfloat16),
            pltpu.VMEM(blk_half, jnp.bfloat16),
            pltpu.SemaphoreType.DMA((6,)),
            pltpu.SemaphoreType.DMA((6,)),
        ],
        compiler_params=pltpu.CompilerParams(collective_id=0),
    )(x, w_mat)


# device time: 46634 ns/iter; 1.0120x vs baseline; 1.0120x over previous
import jax
import jax.numpy as jnp
from jax import lax
from jax.experimental import pallas as pl
from jax.experimental.pallas import tpu as pltpu

N_DEV = 4
M_BLK = 2048 // N_DEV
NQ = 4


def kernel(x, w_mat):
    m, k_per = x.shape
    _, n = w_mat.shape
    qw = n // NQ

    def body(x_ref, w_ref, out_ref, xbf_ref,
             s1A, r1A, s1B, r1B, s2A, r2A, s2B, r2B,
             send1_sems, recv1_sems, send2_sems, recv2_sems):
        p = lax.axis_index("i")
        p1 = jnp.bitwise_xor(p, 1)
        p2 = 3 - p
        bA0 = 3 - p1
        bB0 = jnp.bitwise_xor(p2, 1)

        barrier_sem = pltpu.get_barrier_semaphore()
        for nbr in [p1, p2]:
            pl.semaphore_signal(
                barrier_sem, inc=1,
                device_id=(nbr,), device_id_type=pl.DeviceIdType.MESH,
            )
        pl.semaphore_wait(barrier_sem, 2)

        xbf_ref[:, :] = x_ref[:, :].astype(jnp.bfloat16)
        w_bf = w_ref[:, :].astype(jnp.bfloat16)

        def pq(c, q):
            xc = xbf_ref[pl.ds(c * M_BLK, M_BLK), :]
            wc = w_bf[:, q * qw:(q + 1) * qw]
            return jnp.dot(xc, wc, preferred_element_type=jnp.float32)

        def mk1(src, dst, i, dev):
            return pltpu.make_async_remote_copy(
                src_ref=src, dst_ref=dst,
                send_sem=send1_sems.at[i], recv_sem=recv1_sems.at[i],
                device_id=(dev,), device_id_type=pl.DeviceIdType.MESH,
            )

        def mk2(src, dst, i, dev):
            return pltpu.make_async_remote_copy(
                src_ref=src, dst_ref=dst,
                send_sem=send2_sems.at[i], recv_sem=recv2_sems.at[i],
                device_id=(dev,), device_id_type=pl.DeviceIdType.MESH,
            )

        r1_descs = []
        srcsA = [(0, 0, bA0, 0), (0, 1, bA0, 1), (1, 0, p1, 0), (1, 1, p1, 1)]
        srcsB = [(0, 0, bB0, 2), (0, 1, bB0, 3), (1, 0, p2, 2), (1, 1, p2, 3)]
        dA, dB = [], []
        for i in range(4):
            blkA, subA, cA, qA = srcsA[i]
            s1A[blkA, subA] = pq(cA, qA).astype(jnp.bfloat16)
            d = mk1(s1A.at[blkA, subA], r1A.at[blkA, subA], i, p1)
            d.start()
            dA.append(d)
            blkB, subB, cB, qB = srcsB[i]
            s1B[blkB, subB] = pq(cB, qB).astype(jnp.bfloat16)
            d = mk1(s1B.at[blkB, subB], r1B.at[blkB, subB], 4 + i, p2)
            d.start()
            dB.append(d)
        r1_descs = dA + dB

        pA_fwd = [pq(p2, 0), pq(p2, 1)]
        pB_fwd = [pq(p1, 2), pq(p1, 3)]

        r2_descs = []
        for i in range(2):
            dA[i].wait_recv()
            s2A[i] = (r1A[0, i].astype(jnp.float32) + pA_fwd[i]).astype(jnp.bfloat16)
            d = mk2(s2A.at[i], r2A.at[i], i, p2)
            d.start()
            r2_descs.append(d)
            dB[i].wait_recv()
            s2B[i] = (r1B[0, i].astype(jnp.float32) + pB_fwd[i]).astype(jnp.bfloat16)
            d = mk2(s2B.at[i], r2B.at[i], 2 + i, p1)
            d.start()
            r2_descs.append(d)

        pA_own = [pq(p, 0), pq(p, 1)]
        pB_own = [pq(p, 2), pq(p, 3)]
        dA[2].wait_recv()
        accA = [pA_own[0] + r1A[1, 0].astype(jnp.float32)]
        dA[3].wait_recv()
        accA.append(pA_own[1] + r1A[1, 1].astype(jnp.float32))
        dB[2].wait_recv()
        accB = [pB_own[0] + r1B[1, 0].astype(jnp.float32)]
        dB[3].wait_recv()
        accB.append(pB_own[1] + r1B[1, 1].astype(jnp.float32))

        for i in range(2):
            r2_descs[2 * i].wait_recv()
            out_ref[:, i * qw:(i + 1) * qw] = accA[i] + r2A[i].astype(jnp.float32)
            r2_descs[2 * i + 1].wait_recv()
            out_ref[:, (2 + i) * qw:(3 + i) * qw] = (
                accB[i] + r2B[i].astype(jnp.float32)
            )

        for d in r1_descs + r2_descs:
            d.wait_send()

    blk_q = (M_BLK, qw)
    return pl.pallas_call(
        body,
        out_shape=jax.ShapeDtypeStruct((M_BLK, n), jnp.float32),
        in_specs=[
            pl.BlockSpec(memory_space=pltpu.VMEM),
            pl.BlockSpec(memory_space=pltpu.VMEM),
        ],
        out_specs=pl.BlockSpec(memory_space=pltpu.VMEM),
        scratch_shapes=[
            pltpu.VMEM((m, k_per), jnp.bfloat16),
            pltpu.VMEM((2, 2) + blk_q, jnp.bfloat16),
            pltpu.VMEM((2, 2) + blk_q, jnp.bfloat16),
            pltpu.VMEM((2, 2) + blk_q, jnp.bfloat16),
            pltpu.VMEM((2, 2) + blk_q, jnp.bfloat16),
            pltpu.VMEM((2,) + blk_q, jnp.bfloat16),
            pltpu.VMEM((2,) + blk_q, jnp.bfloat16),
            pltpu.VMEM((2,) + blk_q, jnp.bfloat16),
            pltpu.VMEM((2,) + blk_q, jnp.bfloat16),
            pltpu.SemaphoreType.DMA((8,)),
            pltpu.SemaphoreType.DMA((8,)),
            pltpu.SemaphoreType.DMA((4,)),
            pltpu.SemaphoreType.DMA((4,)),
        ],
        compiler_params=pltpu.CompilerParams(collective_id=0),
    )(x, w_mat)
